# Initial kernel scaffold; baseline (speedup 1.0000x reference)
import jax
import jax.numpy as jnp
from jax.experimental import pallas as pl


def kernel(query, value, W_off, W_att, W_val):
    raise NotImplementedError("write your pallas kernel here")


# speedup vs baseline: 17.7335x; 17.7335x over previous
import functools

import jax
import jax.numpy as jnp
from jax.experimental import pallas as pl
from jax.experimental.pallas import tpu as pltpu

P = 32
K = 8
IN_DIM = 1024
OUT_DIM = 1024

_ROWS = 256


def _prep_body(q_ref, wax_ref, woffx_ref, woffy_ref, s_ref):
    i = pl.program_id(0)
    q = q_ref[...]
    logits = jax.lax.dot_general(
        q, wax_ref[...], (((1,), (1,)), ((), ())),
        preferred_element_type=jnp.float32)
    att = jax.nn.softmax(logits, axis=-1)
    offx = jax.lax.dot_general(
        q, woffx_ref[...], (((1,), (1,)), ((), ())),
        preferred_element_type=jnp.float32)
    offy = jax.lax.dot_general(
        q, woffy_ref[...], (((1,), (1,)), ((), ())),
        preferred_element_type=jnp.float32)

    rows = jax.lax.broadcasted_iota(jnp.int32, (_ROWS, K), 0) + i * _ROWS
    n = jax.lax.rem(rows, jnp.int32(P * P))
    ref_x = (n // P).astype(jnp.float32) / (P - 1.0)
    ref_y = jax.lax.rem(n, jnp.int32(P)).astype(jnp.float32) / (P - 1.0)

    ix = (ref_x + offx) * (P / (P - 1.0)) - 0.5
    iy = (ref_y + offy) * (P / (P - 1.0)) - 0.5
    ix0 = jnp.floor(ix)
    iy0 = jnp.floor(iy)
    wx1 = ix - ix0
    wy1 = iy - iy0

    cell = jax.lax.broadcasted_iota(jnp.int32, (_ROWS, 1, P), 2).astype(jnp.float32)

    def axis_u(i0, w1):
        v0 = ((i0 >= 0) & (i0 <= P - 1)).astype(jnp.float32)
        v1 = ((i0 >= -1) & (i0 <= P - 2)).astype(jnp.float32)
        i0c = jnp.clip(i0, 0.0, P - 1.0)
        i1c = jnp.clip(i0 + 1.0, 0.0, P - 1.0)
        w0 = 1.0 - w1
        u = ((w0 * v0)[:, :, None] * (cell == i0c[:, :, None]).astype(jnp.float32)
             + (w1 * v1)[:, :, None] * (cell == i1c[:, :, None]).astype(jnp.float32))
        return u

    ux = axis_u(ix0, wx1)
    uy = axis_u(iy0, wy1) * att[:, :, None]

    acc = jnp.zeros((_ROWS, P, P), dtype=jnp.float32)
    for k in range(K):
        acc = acc + uy[:, k, :, None] * ux[:, k, None, :]
    s_ref[...] = acc


def _build_s(query, W_off, W_att):
    BN = query.shape[0]
    w_off_x = W_off[0::2]
    w_off_y = W_off[1::2]
    grid = (BN // _ROWS,)
    s3 = pl.pallas_call(
        _prep_body,
        grid=grid,
        in_specs=[
            pl.BlockSpec((_ROWS, IN_DIM), lambda i: (i, 0)),
            pl.BlockSpec((K, IN_DIM), lambda i: (0, 0)),
            pl.BlockSpec((K, IN_DIM), lambda i: (0, 0)),
            pl.BlockSpec((K, IN_DIM), lambda i: (0, 0)),
        ],
        out_specs=pl.BlockSpec((_ROWS, P, P), lambda i: (i, 0, 0)),
        out_shape=jax.ShapeDtypeStruct((BN, P, P), jnp.float32),
    )(query, W_att, w_off_x, w_off_y)
    return s3.reshape(BN, P * P)


_MT = 256


def _mm_body(s_ref, wval_ref, val_ref, out_ref):
    m = jax.lax.dot_general(
        s_ref[...], wval_ref[...], (((1,), (0,)), ((), ())),
        preferred_element_type=jnp.float32)
    out_ref[0] = jax.lax.dot_general(
        m, val_ref[0], (((1,), (1,)), ((), ())),
        preferred_element_type=jnp.float32)


def _matmuls(S, W_val, value):
    B, N, _ = value.shape
    grid = (B, N // _MT)
    out = pl.pallas_call(
        _mm_body,
        grid=grid,
        in_specs=[
            pl.BlockSpec((_MT, P * P), lambda b, t: (b * (N // _MT) + t, 0)),
            pl.BlockSpec((OUT_DIM, IN_DIM), lambda b, t: (0, 0)),
            pl.BlockSpec((1, N, IN_DIM), lambda b, t: (b, 0, 0)),
        ],
        out_specs=pl.BlockSpec((1, _MT, N), lambda b, t: (b, t, 0)),
        out_shape=jax.ShapeDtypeStruct((B, N, N), jnp.float32),
    )(S, W_val, value)
    return out


def kernel(query, value, W_off, W_att, W_val):
    B, N, _ = query.shape
    q2 = query.reshape(B * N, IN_DIM)
    S = _build_s(q2, W_off, W_att)
    return _matmuls(S, W_val, value)


# speedup vs baseline: 40.5459x; 2.2864x over previous
import functools

import jax
import jax.numpy as jnp
from jax import lax
from jax.experimental import pallas as pl
from jax.experimental.pallas import tpu as pltpu
from jax.experimental.pallas import tpu_sc as plsc

P = 32
K = 8
IN_DIM = 1024
OUT_DIM = 1024
M = 4 * K

_ROWS = 256
_NW = 32
_RPW = 128


def _prep_body(q_ref, wax_ref, woffx_ref, woffy_ref, idx_ref, coef_ref):
    i = pl.program_id(0)
    q = q_ref[...]
    logits = jax.lax.dot_general(
        q, wax_ref[...], (((1,), (1,)), ((), ())),
        preferred_element_type=jnp.float32)
    att = jax.nn.softmax(logits, axis=-1)
    offx = jax.lax.dot_general(
        q, woffx_ref[...], (((1,), (1,)), ((), ())),
        preferred_element_type=jnp.float32)
    offy = jax.lax.dot_general(
        q, woffy_ref[...], (((1,), (1,)), ((), ())),
        preferred_element_type=jnp.float32)

    rows = jax.lax.broadcasted_iota(jnp.int32, (_ROWS, K), 0) + i * _ROWS
    n = jax.lax.rem(rows, jnp.int32(P * P))
    ref_x = (n // P).astype(jnp.float32) / (P - 1.0)
    ref_y = jax.lax.rem(n, jnp.int32(P)).astype(jnp.float32) / (P - 1.0)

    ix = (ref_x + offx) * (P / (P - 1.0)) - 0.5
    iy = (ref_y + offy) * (P / (P - 1.0)) - 0.5
    ix0 = jnp.floor(ix)
    iy0 = jnp.floor(iy)
    wx1 = ix - ix0
    wy1 = iy - iy0

    def axis_parts(i0, w1):
        v0 = ((i0 >= 0) & (i0 <= P - 1)).astype(jnp.float32)
        v1 = ((i0 >= -1) & (i0 <= P - 2)).astype(jnp.float32)
        c0 = jnp.clip(i0, 0.0, P - 1.0).astype(jnp.int32)
        c1 = jnp.clip(i0 + 1.0, 0.0, P - 1.0).astype(jnp.int32)
        return (((1.0 - w1) * v0, c0), (w1 * v1, c1))

    xs = axis_parts(ix0, wx1)
    ys = axis_parts(iy0, wy1)
    idx_parts = []
    coef_parts = []
    for wy, cy in ys:
        for wx, cx in xs:
            idx_parts.append(cy * P + cx)
            coef_parts.append(att * wy * wx)
    idx_all = jnp.concatenate(idx_parts, axis=1)
    coef_all = jnp.concatenate(coef_parts, axis=1)
    idx_t = idx_all.T
    coef_t = coef_all.T
    idx_ref[...] = jnp.stack(
        [idx_t[:, 0:_RPW].reshape(M * _RPW),
         idx_t[:, _RPW:].reshape(M * _RPW)])[None]
    coef_ref[...] = jnp.stack(
        [coef_t[:, 0:_RPW].reshape(M * _RPW),
         coef_t[:, _RPW:].reshape(M * _RPW)])[None]


def _prep(query, W_off, W_att):
    BN = query.shape[0]
    w_off_x = W_off[0::2]
    w_off_y = W_off[1::2]
    grid = (BN // _ROWS,)
    idxT, coefT = pl.pallas_call(
        _prep_body,
        grid=grid,
        in_specs=[
            pl.BlockSpec((_ROWS, IN_DIM), lambda i: (i, 0)),
            pl.BlockSpec((K, IN_DIM), lambda i: (0, 0)),
            pl.BlockSpec((K, IN_DIM), lambda i: (0, 0)),
            pl.BlockSpec((K, IN_DIM), lambda i: (0, 0)),
        ],
        out_specs=[
            pl.BlockSpec((1, 2, M * _RPW), lambda i: (i, 0, 0)),
            pl.BlockSpec((1, 2, M * _RPW), lambda i: (i, 0, 0)),
        ],
        out_shape=[
            jax.ShapeDtypeStruct((_NW // 2, 2, M * _RPW), jnp.int32),
            jax.ShapeDtypeStruct((_NW // 2, 2, M * _RPW), jnp.float32),
        ],
    )(query, W_att, w_off_x, w_off_y)
    return idxT.reshape(_NW, M * _RPW), coefT.reshape(_NW, M * _RPW)


_NW = 32
_GROUP = 16


def _sc_scatter_body(idx_hbm, coef_hbm, zeros_hbm, s_hbm,
                     idx_v, coef_v, tile_v):
    wid = lax.axis_index("s") * 2 + lax.axis_index("c")
    pltpu.sync_copy(idx_hbm.at[wid], idx_v)
    pltpu.sync_copy(coef_hbm.at[wid], coef_v)
    lane = lax.broadcasted_iota(jnp.int32, (16,), 0) * (P * P)
    for g in range(_RPW // _GROUP):
        pltpu.sync_copy(zeros_hbm, tile_v)
        for m in range(M):
            iv = idx_v[pl.ds(m * _RPW + g * _GROUP, _GROUP)]
            cv = coef_v[pl.ds(m * _RPW + g * _GROUP, _GROUP)]
            plsc.addupdate_scatter(tile_v, [lane + iv], cv)
        pltpu.sync_copy(
            tile_v,
            s_hbm.at[pl.ds((wid * _RPW + g * _GROUP) * (P * P),
                           _GROUP * P * P)])


def _sc_scatter(idxT, coefT):
    BN = _NW * _RPW
    zeros = jnp.zeros((_GROUP * P * P,), jnp.float32)
    mesh = plsc.VectorSubcoreMesh(core_axis_name="c", subcore_axis_name="s",
                                  num_cores=2, num_subcores=16)
    f = pl.kernel(
        _sc_scatter_body,
        out_type=jax.ShapeDtypeStruct((BN * P * P,), jnp.float32),
        mesh=mesh,
        compiler_params=pltpu.CompilerParams(needs_layout_passes=False),
        scratch_types=[
            pltpu.VMEM((M * _RPW,), jnp.int32),
            pltpu.VMEM((M * _RPW,), jnp.float32),
            pltpu.VMEM((_GROUP * P * P,), jnp.float32),
        ],
    )
    return f(idxT, coefT, zeros).reshape(BN, P * P)


_MT = 256


def _mm_body(s_ref, wval_ref, val_ref, out_ref):
    m = jax.lax.dot_general(
        s_ref[...], wval_ref[...], (((1,), (0,)), ((), ())),
        preferred_element_type=jnp.float32)
    out_ref[0] = jax.lax.dot_general(
        m, val_ref[0], (((1,), (1,)), ((), ())),
        preferred_element_type=jnp.float32)


def _matmuls(S, W_val, value):
    B, N, _ = value.shape
    grid = (B, N // _MT)
    out = pl.pallas_call(
        _mm_body,
        grid=grid,
        in_specs=[
            pl.BlockSpec((_MT, P * P), lambda b, t: (b * (N // _MT) + t, 0)),
            pl.BlockSpec((OUT_DIM, IN_DIM), lambda b, t: (0, 0)),
            pl.BlockSpec((1, N, IN_DIM), lambda b, t: (b, 0, 0)),
        ],
        out_specs=pl.BlockSpec((1, _MT, N), lambda b, t: (b, t, 0)),
        out_shape=jax.ShapeDtypeStruct((B, N, N), jnp.float32),
    )(S, W_val, value)
    return out


def kernel(query, value, W_off, W_att, W_val):
    B, N, _ = query.shape
    q2 = query.reshape(B * N, IN_DIM)
    idxT, coefT = _prep(q2, W_off, W_att)
    S = _sc_scatter(idxT, coefT)
    return _matmuls(S, W_val, value)
